# Initial kernel scaffold; baseline (speedup 1.0000x reference)
import jax
import jax.numpy as jnp
from jax.experimental import pallas as pl


def kernel(source_nodes, destination_nodes, negative_nodes, edge_times, edge_idxs, memory_s, memory_g, W_merge, W_s, W_g):
    raise NotImplementedError("write your pallas kernel here")


# speedup vs baseline: 2.1578x; 2.1578x over previous
import functools

import jax
import jax.numpy as jnp
from jax import lax
from jax.experimental import pallas as pl
from jax.experimental.pallas import tpu as pltpu
from jax.experimental.pallas import tpu_sc as plsc

N_NODES = 100000
MEM_DIM = 200
HALF = MEM_DIM // 2
HPAD = 128
BN = 2000
BB = 2000
NC, NS = 2, 16
NW = NC * NS
CH = 128


def _precompute_body(ms, mg, w1, w2, ws, wg, es, eg):
    t = jnp.tanh(
        jnp.dot(ms[...], w1[...], preferred_element_type=jnp.float32)
        + jnp.dot(mg[...], w2[...], preferred_element_type=jnp.float32))
    es[...] = jnp.dot(t, ws[...], preferred_element_type=jnp.float32)
    eg[...] = jnp.dot(t, wg[...], preferred_element_type=jnp.float32)


def _precompute(ms, mg, w1, w2, ws_pad, wg_pad):
    n = ms.shape[0]
    grid = (n // BN,)
    full = lambda a: pl.BlockSpec(a.shape, lambda i: (0,) * a.ndim)
    return pl.pallas_call(
        _precompute_body,
        grid=grid,
        in_specs=[
            pl.BlockSpec((BN, MEM_DIM), lambda i: (i, 0)),
            pl.BlockSpec((BN, MEM_DIM), lambda i: (i, 0)),
            full(w1), full(w2), full(ws_pad), full(wg_pad),
        ],
        out_specs=[
            pl.BlockSpec((BN, HPAD), lambda i: (i, 0)),
            pl.BlockSpec((BN, HPAD), lambda i: (i, 0)),
        ],
        out_shape=[
            jax.ShapeDtypeStruct((n, HPAD), jnp.float32),
            jax.ShapeDtypeStruct((n, HPAD), jnp.float32),
        ],
    )(ms, mg, w1, w2, ws_pad, wg_pad)


def _make_gather(bp, nch):
    mesh = plsc.VectorSubcoreMesh(core_axis_name="c", subcore_axis_name="s")
    out = jax.ShapeDtypeStruct((bp, HPAD), jnp.float32)

    @functools.partial(
        pl.kernel, mesh=mesh,
        out_type=[out, out, out],
        scratch_types=[
            pltpu.VMEM((CH,), jnp.int32),
            pltpu.VMEM((CH, HPAD), jnp.float32),
            pltpu.SemaphoreType.DMA,
        ],
    )
    def gather_k(es_hbm, eg_hbm, src_hbm, dst_hbm, neg_hbm,
                 os_hbm, od_hbm, on_hbm, idx_v, rows_v, sem):
        wid = lax.axis_index("s") * NC + lax.axis_index("c")
        base = wid * (nch * CH)

        def one_table(table, idxs, outs):
            def chunk(j, carry):
                off = base + j * CH
                pltpu.sync_copy(idxs.at[pl.ds(off, CH)], idx_v)
                pltpu.async_copy(table.at[idx_v], rows_v, sem).wait()
                pltpu.sync_copy(rows_v, outs.at[pl.ds(off, CH)])
                return carry
            lax.fori_loop(0, nch, chunk, 0)

        one_table(es_hbm, src_hbm, os_hbm)
        one_table(eg_hbm, dst_hbm, od_hbm)
        one_table(eg_hbm, neg_hbm, on_hbm)

    return gather_k


def _score_body(a, b, c, pos, neg):
    av = a[...]
    p = jnp.sum(av * b[...], axis=1)
    n = jnp.sum(av * c[...], axis=1)
    pos[...] = (1.0 / (1.0 + jnp.exp(-p)))[None, None, :]
    neg[...] = (1.0 / (1.0 + jnp.exp(-n)))[None, None, :]


def _scores(gs, gd, gn):
    b = gs.shape[0]
    nb = b // BB
    out = jax.ShapeDtypeStruct((nb, 1, BB), jnp.float32)
    pos, neg = pl.pallas_call(
        _score_body,
        grid=(nb,),
        in_specs=[pl.BlockSpec((BB, HPAD), lambda i: (i, 0))] * 3,
        out_specs=[pl.BlockSpec((1, 1, BB), lambda i: (i, 0, 0))] * 2,
        out_shape=[out, out],
    )(gs, gd, gn)
    return pos.reshape(b), neg.reshape(b)


def kernel(source_nodes, destination_nodes, negative_nodes, edge_times,
           edge_idxs, memory_s, memory_g, W_merge, W_s, W_g):
    del edge_times, edge_idxs
    b = source_nodes.shape[0]

    w1 = W_merge[:MEM_DIM]
    w2 = W_merge[MEM_DIM:]
    ws_pad = jnp.zeros((MEM_DIM, HPAD), jnp.float32).at[:, :HALF].set(W_s)
    wg_pad = jnp.zeros((MEM_DIM, HPAD), jnp.float32).at[:, :HALF].set(W_g)

    es, eg = _precompute(memory_s, memory_g, w1, w2, ws_pad, wg_pad)

    nch = -(-b // (NW * CH))
    bp = NW * nch * CH
    pad = bp - b

    def padi(x):
        x = x.astype(jnp.int32)
        return jnp.concatenate([x, jnp.zeros((pad,), jnp.int32)]) if pad else x

    gs, gd, gn = _make_gather(bp, nch)(
        es, eg, padi(source_nodes), padi(destination_nodes),
        padi(negative_nodes))

    return _scores(gs[:b], gd[:b], gn[:b])


# speedup vs baseline: 2.4129x; 1.1182x over previous
import functools

import jax
import jax.numpy as jnp
from jax import lax
from jax.experimental import pallas as pl
from jax.experimental.pallas import tpu as pltpu
from jax.experimental.pallas import tpu_sc as plsc

N_NODES = 100000
MEM_DIM = 200
HALF = MEM_DIM // 2
HPAD = 128
BN = 2000
BB = 2000
NC, NS = 2, 16
NW = NC * NS
CH = 104
NBUF = 8


def _precompute_body(ms, mg, w1, w2, ws, wg, es, eg):
    t = jnp.tanh(
        jnp.dot(ms[...], w1[...], preferred_element_type=jnp.float32)
        + jnp.dot(mg[...], w2[...], preferred_element_type=jnp.float32))
    es[...] = jnp.dot(t, ws[...], preferred_element_type=jnp.float32)
    eg[...] = jnp.dot(t, wg[...], preferred_element_type=jnp.float32)


def _precompute(ms, mg, w1, w2, ws_pad, wg_pad):
    n = ms.shape[0]
    grid = (n // BN,)
    full = lambda a: pl.BlockSpec(a.shape, lambda i: (0,) * a.ndim)
    return pl.pallas_call(
        _precompute_body,
        grid=grid,
        in_specs=[
            pl.BlockSpec((BN, MEM_DIM), lambda i: (i, 0)),
            pl.BlockSpec((BN, MEM_DIM), lambda i: (i, 0)),
            full(w1), full(w2), full(ws_pad), full(wg_pad),
        ],
        out_specs=[
            pl.BlockSpec((BN, HPAD), lambda i: (i, 0)),
            pl.BlockSpec((BN, HPAD), lambda i: (i, 0)),
        ],
        out_shape=[
            jax.ShapeDtypeStruct((n, HPAD), jnp.float32),
            jax.ShapeDtypeStruct((n, HPAD), jnp.float32),
        ],
    )(ms, mg, w1, w2, ws_pad, wg_pad)


def _make_gather(bp):
    mesh = plsc.VectorSubcoreMesh(core_axis_name="c", subcore_axis_name="s")
    rows_per_job = bp // CH // NW

    @functools.partial(
        pl.kernel, mesh=mesh,
        out_type=jax.ShapeDtypeStruct((3 * bp, HPAD), jnp.float32),
        scratch_types=(
            [pltpu.VMEM((2 * rows_per_job, CH), jnp.int32)]
            + [pltpu.VMEM((CH, HPAD), jnp.float32)] * NBUF
            + [pltpu.SemaphoreType.DMA] * (2 * NBUF)
        ),
    )
    def gather_k(es_hbm, eg_hbm, idx_hbm, out_hbm, idx_v, *rest):
        rows = rest[0:NBUF]
        gsem = rest[NBUF:2 * NBUF]
        osem = rest[2 * NBUF:3 * NBUF]
        wid = lax.axis_index("s") * NC + lax.axis_index("c")

        def job(table, row0, nchunks):
            pltpu.sync_copy(idx_hbm.at[pl.ds(row0, nchunks)],
                            idx_v.at[pl.ds(0, nchunks)])

            def body(g2, carry):
                jb = g2 * NBUF
                for h in range(2):
                    hnds = []
                    for i in range(4):
                        buf = h * 4 + i
                        j = jb + buf

                        @pl.when(g2 > 0)
                        def _wait_prev_out(buf=buf):
                            pltpu.make_async_copy(
                                rows[buf], out_hbm.at[pl.ds(0, CH)],
                                osem[buf]).wait()

                        hnds.append(pltpu.async_copy(
                            table.at[idx_v.at[j]], rows[buf], gsem[buf]))
                    for i in range(4):
                        buf = h * 4 + i
                        j = jb + buf
                        hnds[i].wait()
                        pltpu.async_copy(
                            rows[buf],
                            out_hbm.at[pl.ds((row0 + j) * CH, CH)],
                            osem[buf])
                return carry

            lax.fori_loop(0, nchunks // NBUF, body, 0)
            for buf in range(NBUF):
                pltpu.make_async_copy(
                    rows[buf], out_hbm.at[pl.ds(0, CH)], osem[buf]).wait()

        src_rows = bp // CH
        job(es_hbm, wid * rows_per_job, rows_per_job)
        job(eg_hbm, src_rows + wid * 2 * rows_per_job, 2 * rows_per_job)

    return gather_k


def _score_body(a, b, c, pos, neg):
    av = a[...]
    ones = jnp.ones((HPAD, 1), jnp.float32)
    p = jnp.dot(av * b[...], ones, preferred_element_type=jnp.float32)
    n = jnp.dot(av * c[...], ones, preferred_element_type=jnp.float32)
    pos[...] = 1.0 / (1.0 + jnp.exp(-p))
    neg[...] = 1.0 / (1.0 + jnp.exp(-n))


def _scores(gs, gd, gn):
    b = gs.shape[0]
    nb = b // BB
    out = jax.ShapeDtypeStruct((b, 1), jnp.float32)
    pos, neg = pl.pallas_call(
        _score_body,
        grid=(nb,),
        in_specs=[pl.BlockSpec((BB, HPAD), lambda i: (i, 0))] * 3,
        out_specs=[pl.BlockSpec((BB, 1), lambda i: (i, 0))] * 2,
        out_shape=[out, out],
    )(gs, gd, gn)
    return pos.reshape(b), neg.reshape(b)


def kernel(source_nodes, destination_nodes, negative_nodes, edge_times,
           edge_idxs, memory_s, memory_g, W_merge, W_s, W_g):
    del edge_times, edge_idxs
    b = source_nodes.shape[0]

    w1 = W_merge[:MEM_DIM]
    w2 = W_merge[MEM_DIM:]
    ws_pad = jnp.zeros((MEM_DIM, HPAD), jnp.float32).at[:, :HALF].set(W_s)
    wg_pad = jnp.zeros((MEM_DIM, HPAD), jnp.float32).at[:, :HALF].set(W_g)

    es, eg = _precompute(memory_s, memory_g, w1, w2, ws_pad, wg_pad)

    unit = NW * CH * NBUF
    bp = -(-b // unit) * unit
    pad = bp - b

    def padi(x):
        x = x.astype(jnp.int32)
        return jnp.concatenate([x, jnp.zeros((pad,), jnp.int32)]) if pad else x

    idx2d = jnp.concatenate(
        [padi(source_nodes), padi(destination_nodes), padi(negative_nodes)]
    ).reshape(3 * bp // CH, CH)

    out = _make_gather(bp)(es, eg, idx2d)

    return _scores(out[:b], out[bp:bp + b], out[2 * bp:2 * bp + b])


# speedup vs baseline: 4.9114x; 2.0355x over previous
import functools

import jax
import jax.numpy as jnp
from jax import lax
from jax.experimental import pallas as pl
from jax.experimental.pallas import tpu as pltpu
from jax.experimental.pallas import tpu_sc as plsc

N_NODES = 100000
MEM_DIM = 200
HALF = MEM_DIM // 2
HPAD = 128
LANES = 16
BN = 2000
NC, NS = 2, 16
NW = NC * NS
CH = 112
NCHUNK = 14


def _precompute_body(ms, mg, w1, w2, ws, wg, es, eg):
    t = jnp.tanh(
        jnp.dot(ms[...], w1[...], preferred_element_type=jnp.float32)
        + jnp.dot(mg[...], w2[...], preferred_element_type=jnp.float32))
    es[...] = jnp.dot(t, ws[...], preferred_element_type=jnp.float32)
    eg[...] = jnp.dot(t, wg[...], preferred_element_type=jnp.float32)


def _precompute(ms, mg, w1, w2, ws_pad, wg_pad):
    n = ms.shape[0]
    grid = (n // BN,)
    full = lambda a: pl.BlockSpec(a.shape, lambda i: (0,) * a.ndim)
    return pl.pallas_call(
        _precompute_body,
        grid=grid,
        in_specs=[
            pl.BlockSpec((BN, MEM_DIM), lambda i: (i, 0)),
            pl.BlockSpec((BN, MEM_DIM), lambda i: (i, 0)),
            full(w1), full(w2), full(ws_pad), full(wg_pad),
        ],
        out_specs=[
            pl.BlockSpec((BN, HPAD), lambda i: (i, 0)),
            pl.BlockSpec((BN, HPAD), lambda i: (i, 0)),
        ],
        out_shape=[
            jax.ShapeDtypeStruct((n, HPAD), jnp.float32),
            jax.ShapeDtypeStruct((n, HPAD), jnp.float32),
        ],
    )(ms, mg, w1, w2, ws_pad, wg_pad)


def _make_gather_score(bp):
    mesh = plsc.VectorSubcoreMesh(core_axis_name="c", subcore_axis_name="s")
    bw = bp // NW

    @functools.partial(
        pl.kernel, mesh=mesh,
        compiler_params=pltpu.CompilerParams(needs_layout_passes=False),
        out_type=[jax.ShapeDtypeStruct((bp,), jnp.float32)] * 2,
        scratch_types=(
            [pltpu.VMEM((3 * bw,), jnp.int32)]
            + [pltpu.VMEM((CH, HPAD), jnp.float32)] * 6
            + [pltpu.VMEM((bw,), jnp.float32)] * 2
            + [pltpu.SemaphoreType.DMA] * 6
        ),
    )
    def gather_k(es_hbm, eg_hbm, idx_hbm, pos_hbm, neg_hbm,
                 idx_v, a0, b0, c0, a1, b1, c1, pos_v, neg_v, *sems):
        wid = lax.axis_index("s") * NC + lax.axis_index("c")
        sets = ((a0, b0, c0), (a1, b1, c1))
        ssems = (sems[0:3], sems[3:6])

        for t in range(3):
            pltpu.sync_copy(idx_hbm.at[pl.ds(t * bp + wid * bw, bw)],
                            idx_v.at[pl.ds(t * bw, bw)])

        def fire(j, s):
            bufs, sms = sets[s], ssems[s]
            for t, table in enumerate((es_hbm, eg_hbm, eg_hbm)):
                pltpu.async_copy(
                    table.at[idx_v.at[pl.ds(t * bw + j * CH, CH)]],
                    bufs[t], sms[t])

        def consume(j, s):
            bufs, sms = sets[s], ssems[s]
            for t, table in enumerate((es_hbm, eg_hbm, eg_hbm)):
                pltpu.make_async_copy(
                    table.at[idx_v.at[pl.ds(t * bw + j * CH, CH)]],
                    bufs[t], sms[t]).wait()
            ba, bb, bc = bufs
            base = j * CH
            lane = lax.broadcasted_iota(jnp.int32, (LANES,), 0)
            nk = CH // LANES

            def grp(g2, carry):
                z = jnp.zeros((LANES,), jnp.float32)
                accp_v = z
                accn_v = z
                for l in range(LANES):
                    e = g2 * LANES + l
                    ap = z
                    an = z
                    for k in range(nk):
                        a = ba[e, pl.ds(k * LANES, LANES)]
                        ap = ap + a * bb[e, pl.ds(k * LANES, LANES)]
                        an = an + a * bc[e, pl.ds(k * LANES, LANES)]
                    accp_v = jnp.where(lane == l, jnp.sum(ap), accp_v)
                    accn_v = jnp.where(lane == l, jnp.sum(an), accn_v)
                pos_v[pl.ds(base + g2 * LANES, LANES)] = accp_v
                neg_v[pl.ds(base + g2 * LANES, LANES)] = accn_v
                return carry

            lax.fori_loop(0, CH // LANES, grp, 0)

        fire(0, 0)

        def body(g, carry):
            fire(2 * g + 1, 1)
            consume(2 * g, 0)

            @pl.when(g < NCHUNK // 2 - 1)
            def _prefetch():
                fire(2 * g + 2, 0)

            consume(2 * g + 1, 1)
            return carry

        lax.fori_loop(0, NCHUNK // 2, body, 0)

        def sig(i, carry):
            for v in (pos_v, neg_v):
                x = v[pl.ds(i * LANES, LANES)]
                v[pl.ds(i * LANES, LANES)] = 1.0 / (1.0 + jnp.exp(-x))
            return carry

        lax.fori_loop(0, bw // LANES, sig, 0)

        pltpu.sync_copy(pos_v, pos_hbm.at[pl.ds(wid * bw, bw)])
        pltpu.sync_copy(neg_v, neg_hbm.at[pl.ds(wid * bw, bw)])

    return gather_k


def kernel(source_nodes, destination_nodes, negative_nodes, edge_times,
           edge_idxs, memory_s, memory_g, W_merge, W_s, W_g):
    del edge_times, edge_idxs
    b = source_nodes.shape[0]

    w1 = W_merge[:MEM_DIM]
    w2 = W_merge[MEM_DIM:]
    ws_pad = jnp.zeros((MEM_DIM, HPAD), jnp.float32).at[:, :HALF].set(W_s)
    wg_pad = jnp.zeros((MEM_DIM, HPAD), jnp.float32).at[:, :HALF].set(W_g)

    es, eg = _precompute(memory_s, memory_g, w1, w2, ws_pad, wg_pad)

    unit = NW * CH * NCHUNK
    bp = -(-b // unit) * unit
    pad = bp - b

    def padi(x):
        x = x.astype(jnp.int32)
        return jnp.concatenate([x, jnp.zeros((pad,), jnp.int32)]) if pad else x

    idx = jnp.concatenate(
        [padi(source_nodes), padi(destination_nodes), padi(negative_nodes)])

    pos, neg = _make_gather_score(bp)(es, eg, idx)
    return pos[:b], neg[:b]
